# baseline (speedup 1.0000x reference)
import jax
import jax.numpy as jnp
from jax.experimental import pallas as pl

_B, _T, _C, _H, _W = 2, 8, 32, 24, 24
_K = 8192
_N = _B * _T * _H * _W
_BM = 256
_GRID = _N // _BM


def _vq_kernel(zf_ref, cb_ref, idx_ref, feat_ref):
    zf = zf_ref[...]
    cb = cb_ref[...]
    cnorm = jnp.sum(cb * cb, axis=1)
    rnorm = jnp.sum(zf * zf, axis=1, keepdims=True)
    dots = jax.lax.dot_general(
        zf, cb, (((1,), (1,)), ((), ())),
        preferred_element_type=jnp.float32)
    d = rnorm + cnorm[None, :] - 2.0 * dots
    min_d = jnp.min(d, axis=1, keepdims=True)
    ids = jax.lax.broadcasted_iota(jnp.int32, d.shape, 1)
    idx = jnp.min(jnp.where(d == min_d, ids, jnp.int32(_K)), axis=1)
    onehot = (ids == idx[:, None]).astype(jnp.float32)
    feats = jax.lax.dot_general(
        onehot, cb, (((1,), (0,)), ((), ())),
        preferred_element_type=jnp.float32)
    idx_ref[0, 0, :] = idx
    feat_ref[...] = feats


def kernel(z, codebook):
    c = z.shape[1]
    zf = jnp.transpose(z, (0, 2, 3, 1)).reshape(_N, c)
    idx, feats = pl.pallas_call(
        _vq_kernel,
        grid=(_GRID,),
        in_specs=[
            pl.BlockSpec((_BM, _C), lambda i: (i, 0)),
            pl.BlockSpec((_K, _C), lambda i: (0, 0)),
        ],
        out_specs=[
            pl.BlockSpec((1, 1, _BM), lambda i: (i, 0, 0)),
            pl.BlockSpec((_BM, _C), lambda i: (i, 0)),
        ],
        out_shape=[
            jax.ShapeDtypeStruct((_GRID, 1, _BM), jnp.int32),
            jax.ShapeDtypeStruct((_N, _C), jnp.float32),
        ],
    )(zf, codebook)
    L = _H * _W
    indices = idx.reshape(_B, _T, L)
    features = feats.reshape(_B, _T, L, c)
    return indices, features
